# Initial kernel scaffold; baseline (speedup 1.0000x reference)
import jax
import jax.numpy as jnp
from jax.experimental import pallas as pl


def kernel(x, edge_index, edge_weight, W):
    raise NotImplementedError("write your pallas kernel here")


# speedup vs baseline: 5.4270x; 5.4270x over previous
import jax
import jax.numpy as jnp
from jax import lax
from jax.experimental import pallas as pl
from jax.experimental.pallas import tpu as pltpu
from jax.experimental.pallas import tpu_sc as plsc

N = 10000
E = 320000
D = 128

NC = 2
NS = 16
NW = NC * NS
EPW = E // NW
CHUNK = 80
NCHUNK = EPW // CHUNK
P = 10240
RPT = P // NS

_CB = 80


def _matmul_body(x_ref, w_ref, o_ref):
    o_ref[...] = jnp.dot(x_ref[...], w_ref[...],
                         preferred_element_type=jnp.float32)


def _matmul(x, W):
    blk = 1000
    return pl.pallas_call(
        _matmul_body,
        grid=(N // blk,),
        in_specs=[
            pl.BlockSpec((blk, D), lambda i: (i, 0)),
            pl.BlockSpec((D, D), lambda i: (0, 0)),
        ],
        out_specs=pl.BlockSpec((blk, D), lambda i: (i, 0)),
        out_shape=jax.ShapeDtypeStruct((N, D), jnp.float32),
    )(x, W)


def _add_body(a_ref, b_ref, o_ref):
    o_ref[...] = a_ref[...] + b_ref[...]


def _combine(p):
    return pl.pallas_call(
        _add_body,
        grid=(N // _CB,),
        in_specs=[
            pl.BlockSpec((_CB, D), lambda i: (i, 0)),
            pl.BlockSpec((_CB, D), lambda i: (i + P // _CB, 0)),
        ],
        out_specs=pl.BlockSpec((_CB, D), lambda i: (i, 0)),
        out_shape=jax.ShapeDtypeStruct((N, D), jnp.float32),
    )(p, p)


def _spmm_body(h_hbm, src_hbm, dst_hbm, w_hbm, p_hbm,
               src_v, dst_v, w_v, rows_v, acc_sh, sem):
    cid = lax.axis_index("c")
    sid = lax.axis_index("s")
    wid = cid * NS + sid

    pltpu.sync_copy(src_hbm.at[pl.ds(wid * EPW, EPW)], src_v)
    pltpu.sync_copy(dst_hbm.at[wid], dst_v)
    pltpu.sync_copy(w_hbm.at[pl.ds(wid * EPW, EPW)], w_v)

    def _zrow(r, _):
        for k in range(D // 16):
            rows_v[r, pl.ds(k * 16, 16)] = jnp.zeros((16,), jnp.float32)
        return ()
    lax.fori_loop(0, CHUNK, _zrow, ())
    for b in range(RPT // CHUNK):
        pltpu.sync_copy(rows_v, acc_sh.at[pl.ds(sid * RPT + b * CHUNK, CHUNK)])
    plsc.subcore_barrier()

    def _chunk(j, _):
        pltpu.async_copy(h_hbm.at[src_v.at[pl.ds(j * CHUNK, CHUNK)]],
                         rows_v, sem).wait()

        def _group(g, _):
            w16 = w_v[pl.ds(j * CHUNK + g * 16, 16)]
            for l in range(16):
                ws = jnp.broadcast_to(w16[l], (16,))
                e = g * 16 + l
                for k in range(D // 16):
                    sl = pl.ds(k * 16, 16)
                    rows_v[e, sl] = rows_v[e, sl] * ws
            return ()
        lax.fori_loop(0, CHUNK // 16, _group, ())

        pltpu.sync_copy(rows_v, acc_sh.at[dst_v.at[j]], add=True)
        return ()
    lax.fori_loop(0, NCHUNK, _chunk, ())

    plsc.subcore_barrier()
    pltpu.sync_copy(acc_sh.at[pl.ds(sid * RPT, RPT)],
                    p_hbm.at[pl.ds(cid * P + sid * RPT, RPT)])


def _spmm_sc(h, src, dst3d, w):
    mesh = plsc.VectorSubcoreMesh(core_axis_name="c", subcore_axis_name="s")
    return pl.kernel(
        _spmm_body,
        out_type=jax.ShapeDtypeStruct((2 * P, D), jnp.float32),
        mesh=mesh,
        scratch_types=[
            pltpu.VMEM((EPW,), jnp.int32),
            pltpu.VMEM((NCHUNK, CHUNK), jnp.int32),
            pltpu.VMEM((EPW,), jnp.float32),
            pltpu.VMEM((CHUNK, D), jnp.float32),
            pltpu.VMEM_SHARED((P, D), jnp.float32),
            pltpu.SemaphoreType.DMA,
        ],
    )(h, src, dst3d, w)


def kernel(x, edge_index, edge_weight, W):
    src = edge_index[0]
    dst3d = edge_index[1].reshape(NW, NCHUNK, CHUNK)

    h = _matmul(x, W)
    y = _combine(_spmm_sc(h, src, dst3d, edge_weight))
    out = _combine(_spmm_sc(y, src, dst3d, edge_weight))
    return out


# speedup vs baseline: 10.1805x; 1.8759x over previous
import jax
import jax.numpy as jnp
from jax import lax
from jax.experimental import pallas as pl
from jax.experimental.pallas import tpu as pltpu
from jax.experimental.pallas import tpu_sc as plsc

N = 10000
E = 320000
D = 128

NC = 2
NS = 16
NW = NC * NS
EPW = E // NW
CHUNK = 80
NCHUNK = EPW // CHUNK
P = 10240
RPT = P // NS
POFF = 12000

_CB = 2000


def _matmul_body(x_ref, w_ref, o_ref):
    o_ref[...] = jnp.dot(x_ref[...], w_ref[...],
                         preferred_element_type=jnp.float32)


def _matmul(x, W):
    return pl.pallas_call(
        _matmul_body,
        grid=(N // _CB,),
        in_specs=[
            pl.BlockSpec((_CB, D), lambda i: (i, 0)),
            pl.BlockSpec((D, D), lambda i: (0, 0)),
        ],
        out_specs=pl.BlockSpec((_CB, D), lambda i: (i, 0)),
        out_shape=jax.ShapeDtypeStruct((N, D), jnp.float32),
    )(x, W)


def _add_body(a_ref, b_ref, o_ref):
    o_ref[...] = a_ref[...] + b_ref[...]


def _combine(p):
    return pl.pallas_call(
        _add_body,
        grid=(N // _CB,),
        in_specs=[
            pl.BlockSpec((_CB, D), lambda i: (i, 0)),
            pl.BlockSpec((_CB, D), lambda i: (i + POFF // _CB, 0)),
        ],
        out_specs=pl.BlockSpec((_CB, D), lambda i: (i, 0)),
        out_shape=jax.ShapeDtypeStruct((N, D), jnp.float32),
    )(p, p)


def _spmm_body(h_hbm, src_hbm, dst_hbm, w_hbm, p_hbm,
               src_v, w_v, rows_a, rows_b, dst_a, dst_b, acc_sh,
               gsem, ssem, dsem_a, dsem_b):
    cid = lax.axis_index("c")
    sid = lax.axis_index("s")
    wid = cid * NS + sid

    pltpu.sync_copy(src_hbm.at[pl.ds(wid * EPW, EPW)], src_v)
    pltpu.sync_copy(w_hbm.at[pl.ds(wid * EPW, EPW)], w_v)

    def _zrow(r, _):
        for k in range(D // 16):
            rows_a[r, pl.ds(k * 16, 16)] = jnp.zeros((16,), jnp.float32)
        return ()
    lax.fori_loop(0, CHUNK, _zrow, ())
    for b in range(RPT // CHUNK):
        pltpu.sync_copy(rows_a, acc_sh.at[pl.ds(sid * RPT + b * CHUNK, CHUNK)])
    plsc.subcore_barrier()

    def g_copy(j, buf):
        return pltpu.make_async_copy(
            h_hbm.at[src_v.at[pl.ds(j * CHUNK, CHUNK)]], buf, gsem)

    def d_copy(j, dbuf, dsem):
        return pltpu.make_async_copy(
            dst_hbm.at[pl.ds(wid * EPW + j * CHUNK, CHUNK)], dbuf.at[0], dsem)

    def s_wait(rbuf, dbuf):
        pltpu.make_async_copy(rbuf, acc_sh.at[dbuf.at[0]], ssem).wait()

    def s_start(rbuf, dbuf):
        pltpu.async_copy(rbuf, acc_sh.at[dbuf.at[0]], ssem, add=True)

    def scale(j, buf):
        def _group(g, _):
            w16 = w_v[pl.ds(j * CHUNK + g * 16, 16)]
            for l in range(16):
                ws = jnp.broadcast_to(w16[l], (16,))
                e = g * 16 + l
                for k in range(D // 16):
                    sl = pl.ds(k * 16, 16)
                    buf[e, sl] = buf[e, sl] * ws
            return ()
        lax.fori_loop(0, CHUNK // 16, _group, ())

    d_copy(0, dst_a, dsem_a).start()
    g_copy(0, rows_a).start()

    def _pair(jj, _):
        c = 2 * jj
        g_copy(c, rows_a).wait()

        @pl.when(jj > 0)
        def _():
            s_wait(rows_b, dst_b)
        d_copy(c + 1, dst_b, dsem_b).start()
        g_copy(c + 1, rows_b).start()
        scale(c, rows_a)
        d_copy(c, dst_a, dsem_a).wait()
        s_start(rows_a, dst_a)

        g_copy(c + 1, rows_b).wait()
        s_wait(rows_a, dst_a)
        d_copy(c + 2, dst_a, dsem_a).start()
        g_copy(c + 2, rows_a).start()
        scale(c + 1, rows_b)
        d_copy(c + 1, dst_b, dsem_b).wait()
        s_start(rows_b, dst_b)
        return ()
    lax.fori_loop(0, NCHUNK // 2, _pair, ())

    last = NCHUNK - 1
    g_copy(last, rows_a).wait()
    s_wait(rows_b, dst_b)
    scale(last, rows_a)
    d_copy(last, dst_a, dsem_a).wait()
    s_start(rows_a, dst_a)
    s_wait(rows_a, dst_a)

    plsc.subcore_barrier()
    pltpu.sync_copy(acc_sh.at[pl.ds(sid * RPT, RPT)],
                    p_hbm.at[pl.ds(cid * POFF + sid * RPT, RPT)])


def _spmm_sc(h, src, dst, w):
    mesh = plsc.VectorSubcoreMesh(core_axis_name="c", subcore_axis_name="s")
    return pl.kernel(
        _spmm_body,
        out_type=jax.ShapeDtypeStruct((2 * POFF, D), jnp.float32),
        mesh=mesh,
        scratch_types=[
            pltpu.VMEM((EPW,), jnp.int32),
            pltpu.VMEM((EPW,), jnp.float32),
            pltpu.VMEM((CHUNK, D), jnp.float32),
            pltpu.VMEM((CHUNK, D), jnp.float32),
            pltpu.VMEM((1, CHUNK), jnp.int32),
            pltpu.VMEM((1, CHUNK), jnp.int32),
            pltpu.VMEM_SHARED((P, D), jnp.float32),
            pltpu.SemaphoreType.DMA,
            pltpu.SemaphoreType.DMA,
            pltpu.SemaphoreType.DMA,
            pltpu.SemaphoreType.DMA,
        ],
    )(h, src, dst, w)


def kernel(x, edge_index, edge_weight, W):
    src = edge_index[0]
    dst = edge_index[1]

    h = _matmul(x, W)
    y = _combine(_spmm_sc(h, src, dst, edge_weight))
    out = _combine(_spmm_sc(y, src, dst, edge_weight))
    return out


# speedup vs baseline: 11.3091x; 1.1109x over previous
import jax
import jax.numpy as jnp
from jax import lax
from jax.experimental import pallas as pl
from jax.experimental.pallas import tpu as pltpu
from jax.experimental.pallas import tpu_sc as plsc

N = 10000
E = 320000
D = 128

NC = 2
NS = 16
NW = NC * NS
EPW = E // NW
CHUNK = 80
NCHUNK = EPW // CHUNK
MS = 6
P = 10240
RPT = P // NS
POFF = 12000

_CB = 2000


def _matmul_body(x_ref, w_ref, o_ref):
    o_ref[...] = jnp.dot(x_ref[...], w_ref[...],
                         preferred_element_type=jnp.float32)


def _matmul(x, W):
    return pl.pallas_call(
        _matmul_body,
        grid=(N // _CB,),
        in_specs=[
            pl.BlockSpec((_CB, D), lambda i: (i, 0)),
            pl.BlockSpec((D, D), lambda i: (0, 0)),
        ],
        out_specs=pl.BlockSpec((_CB, D), lambda i: (i, 0)),
        out_shape=jax.ShapeDtypeStruct((N, D), jnp.float32),
    )(x, W)


def _add_body(a_ref, b_ref, o_ref):
    o_ref[...] = a_ref[...] + b_ref[...]


def _combine(p):
    return pl.pallas_call(
        _add_body,
        grid=(N // _CB,),
        in_specs=[
            pl.BlockSpec((_CB, D), lambda i: (i, 0)),
            pl.BlockSpec((_CB, D), lambda i: (i + POFF // _CB, 0)),
        ],
        out_specs=pl.BlockSpec((_CB, D), lambda i: (i, 0)),
        out_shape=jax.ShapeDtypeStruct((N, D), jnp.float32),
    )(p, p)


def _spmm_body(h_hbm, src_hbm, dst_hbm, w_hbm, p_hbm,
               rows_a, rows_b, scaled_a, scaled_b,
               src_m, dst_m, w_m, acc_sh,
               gsem_a, gsem_b, ssem_a, ssem_b, msems):
    cid = lax.axis_index("c")
    sid = lax.axis_index("s")
    wid = cid * NS + sid

    def _zrow(r, _):
        for k in range(D // 16):
            scaled_a[r, pl.ds(k * 16, 16)] = jnp.zeros((16,), jnp.float32)
        return ()
    lax.fori_loop(0, CHUNK, _zrow, ())
    for b in range(RPT // CHUNK):
        pltpu.sync_copy(scaled_a,
                        acc_sh.at[pl.ds(sid * RPT + b * CHUNK, CHUNK)])
    plsc.subcore_barrier()

    def m_start(j, slot):
        ed = pl.ds(wid * EPW + j * CHUNK, CHUNK)
        pltpu.async_copy(src_hbm.at[ed], src_m.at[slot], msems.at[slot])
        pltpu.async_copy(dst_hbm.at[ed], dst_m.at[slot], msems.at[slot])
        pltpu.async_copy(w_hbm.at[ed], w_m.at[slot], msems.at[slot])

    def m_wait(j, slot):
        ed = pl.ds(wid * EPW + j * CHUNK, CHUNK)
        pltpu.make_async_copy(src_hbm.at[ed], src_m.at[slot],
                              msems.at[slot]).wait()
        pltpu.make_async_copy(dst_hbm.at[ed], dst_m.at[slot],
                              msems.at[slot]).wait()
        pltpu.make_async_copy(w_hbm.at[ed], w_m.at[slot],
                              msems.at[slot]).wait()

    def g_copy(slot, rbuf, sem):
        return pltpu.make_async_copy(
            h_hbm.at[src_m.at[slot]], rbuf, sem)

    def s_start(obuf, slot, sem):
        pltpu.async_copy(obuf, acc_sh.at[dst_m.at[slot]], sem, add=True)

    def s_wait(obuf, slot, sem):
        pltpu.make_async_copy(obuf, acc_sh.at[dst_m.at[slot]], sem).wait()

    def scale(rbuf, obuf, slot):
        def _group(g, _):
            w16 = w_m[slot, pl.ds(g * 16, 16)]
            for l in range(16):
                ws = jnp.broadcast_to(w16[l], (16,))
                e = g * 16 + l
                for k in range(D // 16):
                    sl = pl.ds(k * 16, 16)
                    obuf[e, sl] = rbuf[e, sl] * ws
            return ()
        lax.fori_loop(0, CHUNK // 16, _group, ())

    for j0 in range(4):
        m_start(j0, j0)
    m_wait(0, 0)
    g_copy(0, rows_a, gsem_a).start()
    m_wait(1, 1)
    g_copy(1, rows_b, gsem_b).start()

    def _pair(jj, _):
        c = 2 * jj
        g_copy(c % MS, rows_a, gsem_a).wait()

        @pl.when(jj > 0)
        def _():
            s_wait(scaled_a, (c - 2) % MS, ssem_a)

        @pl.when(c + 4 < NCHUNK)
        def _():
            m_start(c + 4, (c + 4) % MS)
        scale(rows_a, scaled_a, c % MS)
        m_wait(c + 2, (c + 2) % MS)
        g_copy((c + 2) % MS, rows_a, gsem_a).start()
        s_start(scaled_a, c % MS, ssem_a)

        g_copy((c + 1) % MS, rows_b, gsem_b).wait()

        @pl.when(jj > 0)
        def _():
            s_wait(scaled_b, (c - 1) % MS, ssem_b)

        @pl.when(c + 5 < NCHUNK)
        def _():
            m_start(c + 5, (c + 5) % MS)
        scale(rows_b, scaled_b, (c + 1) % MS)

        @pl.when(c + 3 < NCHUNK)
        def _():
            m_wait(c + 3, (c + 3) % MS)
            g_copy((c + 3) % MS, rows_b, gsem_b).start()
        s_start(scaled_b, (c + 1) % MS, ssem_b)
        return ()
    lax.fori_loop(0, NCHUNK // 2, _pair, ())

    last = NCHUNK - 1
    g_copy(last % MS, rows_a, gsem_a).wait()
    s_wait(scaled_a, (last - 2) % MS, ssem_a)
    scale(rows_a, scaled_a, last % MS)
    s_start(scaled_a, last % MS, ssem_a)
    s_wait(scaled_b, (last - 1) % MS, ssem_b)
    s_wait(scaled_a, last % MS, ssem_a)

    plsc.subcore_barrier()
    pltpu.sync_copy(acc_sh.at[pl.ds(sid * RPT, RPT)],
                    p_hbm.at[pl.ds(cid * POFF + sid * RPT, RPT)])


def _spmm_sc(h, src, dst, w):
    mesh = plsc.VectorSubcoreMesh(core_axis_name="c", subcore_axis_name="s")
    return pl.kernel(
        _spmm_body,
        out_type=jax.ShapeDtypeStruct((2 * POFF, D), jnp.float32),
        mesh=mesh,
        compiler_params=pltpu.CompilerParams(needs_layout_passes=False),
        scratch_types=[
            pltpu.VMEM((CHUNK, D), jnp.float32),
            pltpu.VMEM((CHUNK, D), jnp.float32),
            pltpu.VMEM((CHUNK, D), jnp.float32),
            pltpu.VMEM((CHUNK, D), jnp.float32),
            pltpu.VMEM((MS, CHUNK), jnp.int32),
            pltpu.VMEM((MS, CHUNK), jnp.int32),
            pltpu.VMEM((MS, CHUNK), jnp.float32),
            pltpu.VMEM_SHARED((P, D), jnp.float32),
            pltpu.SemaphoreType.DMA,
            pltpu.SemaphoreType.DMA,
            pltpu.SemaphoreType.DMA,
            pltpu.SemaphoreType.DMA,
            pltpu.SemaphoreType.DMA((MS,)),
        ],
    )(h, src, dst, w)


def kernel(x, edge_index, edge_weight, W):
    src = edge_index[0]
    dst = edge_index[1]

    h = _matmul(x, W)
    y = _combine(_spmm_sc(h, src, dst, edge_weight))
    out = _combine(_spmm_sc(y, src, dst, edge_weight))
    return out
